# baseline (device time: 21031 ns/iter reference)
import jax
import jax.numpy as jnp
from jax import lax
from jax.experimental import pallas as pl
from jax.experimental.pallas import tpu as pltpu

N_CHUNKS = 8


def kernel(x):
    _, m, n_full = x.shape
    n_half = n_full // 2
    mc = m // N_CHUNKS

    def body(
        x_ref,
        out_ref,
        out_stage,
        send_q,
        recv_q,
        amax_send,
        amax_recv,
        send_sems,
        recv_sems,
        store_sems,
        amax_send_sem,
        amax_recv_sem,
    ):
        mx = lax.axis_index("x")
        my = lax.axis_index("y")
        mz = lax.axis_index("z")
        peer = (1 - mx, my, mz)
        peer_cols = pl.ds((1 - mx) * n_half, n_half)
        my_cols = pl.ds(mx * n_half, n_half)

        amaxes = []
        for i in range(N_CHUNKS):
            rows = pl.ds(i * mc, mc)
            amax = jnp.maximum(
                jnp.max(jnp.abs(x_ref[0, rows, peer_cols])), 1e-30
            )
            amax_send[i, :] = jnp.full((128,), amax, jnp.float32)
            amaxes.append(amax)

        barrier_sem = pltpu.get_barrier_semaphore()
        pl.semaphore_signal(
            barrier_sem, inc=1, device_id=peer,
            device_id_type=pl.DeviceIdType.MESH,
        )
        pl.semaphore_wait(barrier_sem, 1)

        amax_rdma = pltpu.make_async_remote_copy(
            src_ref=amax_send,
            dst_ref=amax_recv,
            send_sem=amax_send_sem,
            recv_sem=amax_recv_sem,
            device_id=peer,
            device_id_type=pl.DeviceIdType.MESH,
        )
        amax_rdma.start()

        rdmas = []
        for i in range(N_CHUNKS):
            rows = pl.ds(i * mc, mc)
            send_q[rows, :] = jnp.round(
                x_ref[0, rows, peer_cols] * (127.0 / amaxes[i])
            ).astype(jnp.int8)
            r = pltpu.make_async_remote_copy(
                src_ref=send_q.at[rows, :],
                dst_ref=recv_q.at[rows, :],
                send_sem=send_sems.at[i],
                recv_sem=recv_sems.at[i],
                device_id=peer,
                device_id_type=pl.DeviceIdType.MESH,
            )
            r.start()
            rdmas.append(r)

        amax_rdma.wait_recv()
        stores = []
        for i in range(N_CHUNKS):
            rows = pl.ds(i * mc, mc)
            rdmas[i].wait_recv()
            out_stage[rows, :] = (
                x_ref[0, rows, my_cols]
                + recv_q[rows, :].astype(jnp.float32)
                * (amax_recv[i:i + 1, 0:1] * (1.0 / 127.0))
            )
            st = pltpu.make_async_copy(
                out_stage.at[rows, :],
                out_ref.at[rows, :],
                store_sems.at[i],
            )
            st.start()
            stores.append(st)

        amax_rdma.wait_send()
        for i in range(N_CHUNKS):
            rdmas[i].wait_send()
            stores[i].wait()

    return pl.pallas_call(
        body,
        out_shape=jax.ShapeDtypeStruct((m, n_half), x.dtype),
        in_specs=[pl.BlockSpec(memory_space=pltpu.VMEM)],
        out_specs=pl.BlockSpec(memory_space=pl.ANY),
        scratch_shapes=[
            pltpu.VMEM((m, n_half), jnp.float32),
            pltpu.VMEM((m, n_half), jnp.int8),
            pltpu.VMEM((m, n_half), jnp.int8),
            pltpu.VMEM((N_CHUNKS, 128), jnp.float32),
            pltpu.VMEM((N_CHUNKS, 128), jnp.float32),
            pltpu.SemaphoreType.DMA((N_CHUNKS,)),
            pltpu.SemaphoreType.DMA((N_CHUNKS,)),
            pltpu.SemaphoreType.DMA((N_CHUNKS,)),
            pltpu.SemaphoreType.DMA,
            pltpu.SemaphoreType.DMA,
        ],
        compiler_params=pltpu.CompilerParams(collective_id=0),
    )(x)


# device time: 20917 ns/iter; 1.0055x vs baseline; 1.0055x over previous
import jax
import jax.numpy as jnp
from jax import lax
from jax.experimental import pallas as pl
from jax.experimental.pallas import tpu as pltpu

N_CHUNKS = 8


def kernel(x):
    _, m, n_full = x.shape
    n_half = n_full // 2
    mc = m // N_CHUNKS

    def body(
        x_ref,
        out_ref,
        send_q,
        recv_q,
        amax_send,
        amax_recv,
        send_sems,
        recv_sems,
        amax_send_sem,
        amax_recv_sem,
    ):
        mx = lax.axis_index("x")
        my = lax.axis_index("y")
        mz = lax.axis_index("z")
        peer = (1 - mx, my, mz)
        peer_cols = pl.ds((1 - mx) * n_half, n_half)
        my_cols = pl.ds(mx * n_half, n_half)

        barrier_sem = pltpu.get_barrier_semaphore()
        pl.semaphore_signal(
            barrier_sem, inc=1, device_id=peer,
            device_id_type=pl.DeviceIdType.MESH,
        )
        pl.semaphore_wait(barrier_sem, 1)

        amaxes = []
        for i in range(N_CHUNKS):
            rows = pl.ds(i * mc, mc)
            amax = jnp.maximum(
                jnp.max(jnp.abs(x_ref[0, rows, peer_cols])), 1e-30
            )
            amax_send[i, :] = jnp.full((128,), amax, jnp.float32)
            amaxes.append(amax)

        amax_rdma = pltpu.make_async_remote_copy(
            src_ref=amax_send,
            dst_ref=amax_recv,
            send_sem=amax_send_sem,
            recv_sem=amax_recv_sem,
            device_id=peer,
            device_id_type=pl.DeviceIdType.MESH,
        )
        amax_rdma.start()

        rdmas = []
        for i in range(N_CHUNKS):
            rows = pl.ds(i * mc, mc)
            send_q[rows, :] = jnp.round(
                x_ref[0, rows, peer_cols] * (127.0 / amaxes[i])
            ).astype(jnp.int8)
            r = pltpu.make_async_remote_copy(
                src_ref=send_q.at[rows, :],
                dst_ref=recv_q.at[rows, :],
                send_sem=send_sems.at[i],
                recv_sem=recv_sems.at[i],
                device_id=peer,
                device_id_type=pl.DeviceIdType.MESH,
            )
            r.start()
            rdmas.append(r)

        amax_rdma.wait_recv()
        for i in range(N_CHUNKS):
            rows = pl.ds(i * mc, mc)
            rdmas[i].wait_recv()
            out_ref[rows, :] = (
                x_ref[0, rows, my_cols]
                + recv_q[rows, :].astype(jnp.float32)
                * (amax_recv[i:i + 1, 0:1] * (1.0 / 127.0))
            )

        amax_rdma.wait_send()
        for i in range(N_CHUNKS):
            rdmas[i].wait_send()

    return pl.pallas_call(
        body,
        out_shape=jax.ShapeDtypeStruct((m, n_half), x.dtype),
        in_specs=[pl.BlockSpec(memory_space=pltpu.VMEM)],
        out_specs=pl.BlockSpec(memory_space=pltpu.VMEM),
        scratch_shapes=[
            pltpu.VMEM((m, n_half), jnp.int8),
            pltpu.VMEM((m, n_half), jnp.int8),
            pltpu.VMEM((N_CHUNKS, 128), jnp.float32),
            pltpu.VMEM((N_CHUNKS, 128), jnp.float32),
            pltpu.SemaphoreType.DMA((N_CHUNKS,)),
            pltpu.SemaphoreType.DMA((N_CHUNKS,)),
            pltpu.SemaphoreType.DMA,
            pltpu.SemaphoreType.DMA,
        ],
        compiler_params=pltpu.CompilerParams(collective_id=0),
    )(x)
